# baseline (speedup 1.0000x reference)
import functools
import math

import jax
import jax.numpy as jnp
import numpy as np
from jax.experimental import pallas as pl
from jax.experimental.pallas import tpu as pltpu
from jax.experimental.pallas import tpu_sc as plsc

VOCAB = 100000
EMB_DIM = 128
D_MODEL = 768
B = 1024
S = 50
NTOK = B * S
SCALE = math.sqrt(EMB_DIM)

SPLITS = (50,)

NC, NS = 2, 16
NW = NC * NS
CH = 80
NBUF = 6

TOK_BLK = 5120
SPB = TOK_BLK // B


def _positional_encoding_np(seq_len, d):
    pos = np.arange(seq_len, dtype=np.float32)[:, None]
    div = np.exp(np.arange(0, d, 2, dtype=np.float32) * (-math.log(10000.0) / d))
    pe = np.zeros((seq_len, d), dtype=np.float32)
    pe[:, 0::2] = np.sin(pos * div)
    pe[:, 1::2] = np.cos(pos * div)
    return pe


_PE = _positional_encoding_np(S, EMB_DIM).reshape(S // SPB, SPB, EMB_DIM)


@functools.cache
def _sc_gather(ntok):
    rows_per_w = ntok // NW
    nch = rows_per_w // CH

    def body(idx_hbm, table_hbm, out_hbm, idx_v, bufs, obufs, gsems, ssems):
        cid = jax.lax.axis_index("c")
        sid = jax.lax.axis_index("s")
        wid = sid * NC + cid
        base = wid * rows_per_w
        pltpu.sync_copy(idx_hbm.at[wid], idx_v)

        def gather(j):
            return pltpu.async_copy(table_hbm.at[idx_v.at[j]], bufs[j % NBUF],
                                    gsems[j % NBUF])

        def convert(j):
            gbuf, obuf = bufs[j % NBUF], obufs[j % NBUF]

            @plsc.parallel_loop(0, CH // 2, step=1, unroll=8)
            def row(p):
                for jc in range(EMB_DIM // 16):
                    sl = pl.ds(jc * 16, 16)
                    a = jax.lax.bitcast_convert_type(gbuf[2 * p, sl],
                                                     jnp.int32)
                    bb = jax.lax.bitcast_convert_type(gbuf[2 * p + 1, sl],
                                                      jnp.int32)
                    obuf[p, sl] = (jax.lax.shift_right_logical(a, 16)
                                   | (bb & jnp.int32(-65536)))

        base2 = wid * (rows_per_w // 2)

        def store(j):
            return pltpu.async_copy(
                obufs[j % NBUF],
                out_hbm.at[pl.ds(base2 + j * (CH // 2), CH // 2)],
                ssems[j % NBUF])

        pend_g = [None] * NBUF
        pend_s = [None] * NBUF
        for k in range(min(NBUF - 1, nch)):
            pend_g[k % NBUF] = gather(k)
        for j in range(nch):
            pend_g[j % NBUF].wait()
            convert(j)
            pend_s[j % NBUF] = store(j)
            nxt = j + NBUF - 1
            if nxt < nch:
                if pend_s[nxt % NBUF] is not None:
                    pend_s[nxt % NBUF].wait()
                    pend_s[nxt % NBUF] = None
                pend_g[nxt % NBUF] = gather(nxt)
        for k in range(NBUF):
            if pend_s[k] is not None:
                pend_s[k].wait()

    return pl.kernel(
        body,
        out_type=jax.ShapeDtypeStruct((ntok // 2, EMB_DIM), jnp.int32),
        mesh=plsc.VectorSubcoreMesh(core_axis_name="c", subcore_axis_name="s",
                                    num_cores=NC, num_subcores=NS),
        scratch_types=[
            pltpu.VMEM((nch, CH), jnp.int32),
            [pltpu.VMEM((CH, EMB_DIM), jnp.float32) for _ in range(NBUF)],
            [pltpu.VMEM((CH // 2, EMB_DIM), jnp.int32) for _ in range(NBUF)],
            [pltpu.SemaphoreType.DMA for _ in range(NBUF)],
            [pltpu.SemaphoreType.DMA for _ in range(NBUF)],
        ],
    )


def _mm_body(x_ref, pe_ref, w_ref, b_ref, o_ref):
    xb = pltpu.bitcast(x_ref[...], jnp.bfloat16)
    x3 = (xb.astype(jnp.float32).reshape(SPB, B, EMB_DIM) * SCALE
          + pe_ref[0][:, None, :])
    x = x3.reshape(TOK_BLK, EMB_DIM)
    o_ref[...] = jax.lax.dot_general(
        x.astype(jnp.bfloat16), w_ref[...].astype(jnp.bfloat16),
        (((1,), (1,)), ((), ())),
        preferred_element_type=jnp.float32) + b_ref[...]


def _mm_body_acc(acc_ref, x_ref, pe_ref, w_ref, b_ref, o_ref):
    del acc_ref
    _mm_body(x_ref, pe_ref, w_ref, b_ref, o_ref)


def _tc_project_chunk(off, steps, prev, gathered_k, pe, W, b2):
    in_specs = [
        pl.BlockSpec((TOK_BLK // 2, EMB_DIM), lambda i: (i, 0)),
        pl.BlockSpec((1, SPB, EMB_DIM), lambda i: (i + off, 0, 0)),
        pl.BlockSpec((D_MODEL, EMB_DIM), lambda i: (0, 0)),
        pl.BlockSpec((1, D_MODEL), lambda i: (0, 0)),
    ]
    out_spec = pl.BlockSpec((TOK_BLK, D_MODEL), lambda i: (i + off, 0))
    out_shape = jax.ShapeDtypeStruct((NTOK, D_MODEL), jnp.float32)
    params = pltpu.CompilerParams(dimension_semantics=("arbitrary",))
    if prev is None:
        return pl.pallas_call(
            _mm_body, grid=(steps,), in_specs=in_specs, out_specs=out_spec,
            out_shape=out_shape, compiler_params=params,
        )(gathered_k, pe, W, b2)
    return pl.pallas_call(
        _mm_body_acc, grid=(steps,),
        in_specs=[pl.BlockSpec(memory_space=pl.ANY)] + in_specs,
        out_specs=out_spec, out_shape=out_shape,
        input_output_aliases={0: 0}, compiler_params=params,
    )(prev, gathered_k, pe, W, b2)


def kernel(output, emb_table, W, b):
    flat = output.astype(jnp.int32).T.reshape(NTOK)
    pe = jnp.asarray(_PE)
    b2 = b.reshape(1, D_MODEL)
    gs = []
    off = 0
    for s_k in SPLITS:
        ntok = s_k * B
        idx3 = jax.lax.slice(flat, (off * B,), (off * B + ntok,)).reshape(
            NW, ntok // NW // CH, CH)
        gs.append(_sc_gather(ntok)(idx3, emb_table))
        off += s_k
    out = None
    off = 0
    for s_k, g in zip(SPLITS, gs):
        steps = s_k * B // TOK_BLK
        out = _tc_project_chunk(off * B // TOK_BLK, steps, out, g, pe, W, b2)
        off += s_k
    return out.reshape(S, B, D_MODEL).transpose(1, 0, 2)


# speedup vs baseline: 1.0363x; 1.0363x over previous
import functools
import math

import jax
import jax.numpy as jnp
import numpy as np
from jax.experimental import pallas as pl
from jax.experimental.pallas import tpu as pltpu
from jax.experimental.pallas import tpu_sc as plsc

VOCAB = 100000
EMB_DIM = 128
D_MODEL = 768
B = 1024
S = 50
NTOK = B * S
SCALE = math.sqrt(EMB_DIM)

SPLITS = (50,)

NC, NS = 2, 16
NW = NC * NS
CH = 80
NBUF = 8

TOK_BLK = 5120
SPB = TOK_BLK // B


def _positional_encoding_np(seq_len, d):
    pos = np.arange(seq_len, dtype=np.float32)[:, None]
    div = np.exp(np.arange(0, d, 2, dtype=np.float32) * (-math.log(10000.0) / d))
    pe = np.zeros((seq_len, d), dtype=np.float32)
    pe[:, 0::2] = np.sin(pos * div)
    pe[:, 1::2] = np.cos(pos * div)
    return pe


_PE = _positional_encoding_np(S, EMB_DIM).reshape(S // SPB, SPB, EMB_DIM)


@functools.cache
def _sc_gather(ntok):
    rows_per_w = ntok // NW
    nch = rows_per_w // CH

    def body(idx_hbm, table_hbm, out_hbm, idx_v, bufs, gsems, ssems):
        cid = jax.lax.axis_index("c")
        sid = jax.lax.axis_index("s")
        wid = sid * NC + cid
        base = wid * rows_per_w
        pltpu.sync_copy(idx_hbm.at[wid], idx_v)

        def gather(j):
            return pltpu.async_copy(table_hbm.at[idx_v.at[j]], bufs[j % NBUF],
                                    gsems[j % NBUF])

        def store(j):
            return pltpu.async_copy(bufs[j % NBUF],
                                    out_hbm.at[pl.ds(base + j * CH, CH)],
                                    ssems[j % NBUF])

        pend_g = [None] * NBUF
        pend_s = [None] * NBUF
        for k in range(min(NBUF - 1, nch)):
            pend_g[k % NBUF] = gather(k)
        for j in range(nch):
            pend_g[j % NBUF].wait()
            pend_s[j % NBUF] = store(j)
            nxt = j + NBUF - 1
            if nxt < nch:
                if pend_s[nxt % NBUF] is not None:
                    pend_s[nxt % NBUF].wait()
                    pend_s[nxt % NBUF] = None
                pend_g[nxt % NBUF] = gather(nxt)
        for k in range(NBUF):
            if pend_s[k] is not None:
                pend_s[k].wait()

    return pl.kernel(
        body,
        out_type=jax.ShapeDtypeStruct((ntok, EMB_DIM), jnp.float32),
        mesh=plsc.VectorSubcoreMesh(core_axis_name="c", subcore_axis_name="s",
                                    num_cores=NC, num_subcores=NS),
        scratch_types=[
            pltpu.VMEM((nch, CH), jnp.int32),
            [pltpu.VMEM((CH, EMB_DIM), jnp.float32) for _ in range(NBUF)],
            [pltpu.SemaphoreType.DMA for _ in range(NBUF)],
            [pltpu.SemaphoreType.DMA for _ in range(NBUF)],
        ],
    )


def _mm_body(x_ref, pe_ref, w_ref, b_ref, o_ref):
    x3 = x_ref[...].reshape(SPB, B, EMB_DIM) * SCALE + pe_ref[0][:, None, :]
    x = x3.reshape(TOK_BLK, EMB_DIM)
    o_ref[...] = jax.lax.dot_general(
        x.astype(jnp.bfloat16), w_ref[...].astype(jnp.bfloat16),
        (((1,), (1,)), ((), ())),
        preferred_element_type=jnp.float32) + b_ref[...]


def _mm_body_acc(acc_ref, x_ref, pe_ref, w_ref, b_ref, o_ref):
    del acc_ref
    _mm_body(x_ref, pe_ref, w_ref, b_ref, o_ref)


def _tc_project_chunk(off, steps, prev, gathered_k, pe, W, b2):
    in_specs = [
        pl.BlockSpec((TOK_BLK, EMB_DIM), lambda i: (i, 0)),
        pl.BlockSpec((1, SPB, EMB_DIM), lambda i: (i + off, 0, 0)),
        pl.BlockSpec((D_MODEL, EMB_DIM), lambda i: (0, 0)),
        pl.BlockSpec((1, D_MODEL), lambda i: (0, 0)),
    ]
    out_spec = pl.BlockSpec((TOK_BLK, D_MODEL), lambda i: (i + off, 0))
    out_shape = jax.ShapeDtypeStruct((NTOK, D_MODEL), jnp.float32)
    params = pltpu.CompilerParams(dimension_semantics=("arbitrary",))
    if prev is None:
        return pl.pallas_call(
            _mm_body, grid=(steps,), in_specs=in_specs, out_specs=out_spec,
            out_shape=out_shape, compiler_params=params,
        )(gathered_k, pe, W, b2)
    return pl.pallas_call(
        _mm_body_acc, grid=(steps,),
        in_specs=[pl.BlockSpec(memory_space=pl.ANY)] + in_specs,
        out_specs=out_spec, out_shape=out_shape,
        input_output_aliases={0: 0}, compiler_params=params,
    )(prev, gathered_k, pe, W, b2)


def kernel(output, emb_table, W, b):
    flat = output.astype(jnp.int32).T.reshape(NTOK)
    pe = jnp.asarray(_PE)
    b2 = b.reshape(1, D_MODEL)
    gs = []
    off = 0
    for s_k in SPLITS:
        ntok = s_k * B
        idx3 = jax.lax.slice(flat, (off * B,), (off * B + ntok,)).reshape(
            NW, ntok // NW // CH, CH)
        gs.append(_sc_gather(ntok)(idx3, emb_table))
        off += s_k
    out = None
    off = 0
    for s_k, g in zip(SPLITS, gs):
        steps = s_k * B // TOK_BLK
        out = _tc_project_chunk(off * B // TOK_BLK, steps, out, g, pe, W, b2)
        off += s_k
    return out.reshape(S, B, D_MODEL).transpose(1, 0, 2)
